# baseline (speedup 1.0000x reference)
import functools

import numpy as np
import jax
import jax.numpy as jnp
from jax import lax
from jax.experimental import pallas as pl
from jax.experimental.pallas import tpu as pltpu
from jax.experimental.pallas import tpu_sc as plsc

_IMAGE = 200
_PC = 50
_NPTS = _PC * _PC
_LANES = 16
_PAD = 2512
_NFULL = _NPTS // _LANES
_NC, _NS = 2, 16
_NW = _NC * _NS


def _build_tables():
    x1 = np.linspace(0.0, 1.0, _PC).astype(np.float32)
    xpc = np.tile(x1, _PC)
    ypc = np.repeat(x1, _PC)
    gx = xpc * np.float32(_IMAGE - 1)
    gy = ypc * np.float32(_IMAGE - 1)
    ix0 = np.clip(np.floor(gx).astype(np.int32), 0, _IMAGE - 1)
    iy0 = np.clip(np.floor(gy).astype(np.int32), 0, _IMAGE - 1)
    ix1 = np.clip(ix0 + 1, 0, _IMAGE - 1)
    iy1 = np.clip(iy0 + 1, 0, _IMAGE - 1)
    wx = gx - ix0.astype(np.float32)
    wy = gy - iy0.astype(np.float32)
    idx = np.zeros((4, _PAD), np.int32)
    wts = np.zeros((4, _PAD), np.float32)
    idx[0, :_NPTS] = iy0
    idx[1, :_NPTS] = iy1
    idx[2, :_NPTS] = ix0
    idx[3, :_NPTS] = ix1
    wts[0, :_NPTS] = (1.0 - wx) * (1.0 - wy)
    wts[1, :_NPTS] = wx * (1.0 - wy)
    wts[2, :_NPTS] = (1.0 - wx) * wy
    wts[3, :_NPTS] = wx * wy
    return idx, wts


_IDX_TABLE, _WTS_TABLE = _build_tables()


def _make_sampler(n_frames):
    frames_per_w = n_frames // _NW
    opts = frames_per_w * _NPTS

    mesh = plsc.VectorSubcoreMesh(core_axis_name="c", subcore_axis_name="s")

    @functools.partial(
        pl.kernel,
        mesh=mesh,
        out_type=(
            jax.ShapeDtypeStruct((n_frames * _NPTS,), jnp.float32),
            jax.ShapeDtypeStruct((n_frames * _NPTS,), jnp.float32),
        ),
        compiler_params=pltpu.CompilerParams(needs_layout_passes=False),
        scratch_types=[
            pltpu.VMEM((4, _PAD), jnp.int32),
            pltpu.VMEM((4, _PAD), jnp.float32),
            pltpu.VMEM((_IMAGE, _IMAGE), jnp.float32),
            pltpu.VMEM((_IMAGE, _IMAGE), jnp.float32),
            pltpu.VMEM((2 * _NPTS,), jnp.float32),
            pltpu.SemaphoreType.DMA,
            pltpu.SemaphoreType.DMA,
        ],
    )
    def sampler(frames_hbm, idx_hbm, wts_hbm, out0_hbm, out1_hbm,
                idx_v, wts_v, fbuf0, fbuf1, ovec, sem0, sem1):
        wid = lax.axis_index("s") * _NC + lax.axis_index("c")
        base = wid * frames_per_w
        pltpu.sync_copy(idx_hbm, idx_v)
        pltpu.sync_copy(wts_hbm, wts_v)
        bufs = (fbuf0, fbuf1)
        sems = (sem0, sem1)
        tail_lane = lax.iota(jnp.int32, _LANES)
        tail_mask = tail_lane < (_NPTS - _NFULL * _LANES)
        nxt = pltpu.async_copy(frames_hbm.at[base], fbuf0, sem0)
        for k in range(frames_per_w):
            cur_buf = bufs[k % 2]
            cur_cp = nxt
            if k + 1 < frames_per_w:
                nxt = pltpu.async_copy(
                    frames_hbm.at[base + k + 1], bufs[(k + 1) % 2],
                    sems[(k + 1) % 2])
            cur_cp.wait()
            obase = (k % 2) * _NPTS

            @plsc.parallel_loop(0, _NFULL, unroll=6)
            def _chunk(i):
                sl = pl.ds(i * _LANES, _LANES)
                v0 = plsc.load_gather(cur_buf, [idx_v[0, sl], idx_v[2, sl]])
                v1 = plsc.load_gather(cur_buf, [idx_v[0, sl], idx_v[3, sl]])
                v2 = plsc.load_gather(cur_buf, [idx_v[1, sl], idx_v[2, sl]])
                v3 = plsc.load_gather(cur_buf, [idx_v[1, sl], idx_v[3, sl]])
                ovec[pl.ds(obase + i * _LANES, _LANES)] = (
                    v0 * wts_v[0, sl] + v1 * wts_v[1, sl]
                    + v2 * wts_v[2, sl] + v3 * wts_v[3, sl])

            sl = pl.ds(_NFULL * _LANES, _LANES)
            v0 = plsc.load_gather(cur_buf, [idx_v[0, sl], idx_v[2, sl]])
            v1 = plsc.load_gather(cur_buf, [idx_v[0, sl], idx_v[3, sl]])
            v2 = plsc.load_gather(cur_buf, [idx_v[1, sl], idx_v[2, sl]])
            v3 = plsc.load_gather(cur_buf, [idx_v[1, sl], idx_v[3, sl]])
            tail = (v0 * wts_v[0, sl] + v1 * wts_v[1, sl]
                    + v2 * wts_v[2, sl] + v3 * wts_v[3, sl])
            plsc.store_scatter(
                ovec, [obase + _NFULL * _LANES + tail_lane], tail,
                mask=tail_mask)

            if k % 2 == 1:
                out_lo = (base + k - 1) * _NPTS
                pltpu.sync_copy(ovec, out0_hbm.at[pl.ds(out_lo, 2 * _NPTS)])
                pltpu.sync_copy(ovec, out1_hbm.at[pl.ds(out_lo, 2 * _NPTS)])

    return sampler


def kernel(input):
    B, T, C, H, W = input.shape
    n_frames = B * T * C
    frames = input.reshape(n_frames, H, W)
    sampler = _make_sampler(n_frames)
    out0, out1 = sampler(frames, jnp.asarray(_IDX_TABLE),
                         jnp.asarray(_WTS_TABLE))
    res0 = out0.reshape(B, T, C, _NPTS)
    res1 = out1.reshape(B, T, C, _NPTS)
    return (res0, res1)
